# baseline (speedup 1.0000x reference)
import functools

import jax
import jax.numpy as jnp
from jax import lax
from jax.experimental import pallas as pl
from jax.experimental.pallas import tpu as pltpu
from jax.experimental.pallas import tpu_sc as plsc

N_NODES_ = 10000
N_EDGES_ = 320000
D_ = 128

NUM_CORES = 2
NUM_SUBCORES = 16
NUM_TILES = NUM_CORES * NUM_SUBCORES
EDGES_PER_TILE = N_EDGES_ // NUM_TILES
CHUNK = 80
CHUNKS_PER_TILE = EDGES_PER_TILE // CHUNK
ROWS_PER_TILE = 624
ZCHUNK = 208
TAIL_BASE = ROWS_PER_TILE * NUM_SUBCORES
TAIL_ROWS = N_NODES_ - TAIL_BASE


def _sc_body(x_hbm, row_hbm, col_hbm, out_hbm, colbig, rowbig,
             colv0, colv1, rowv0, rowv1, buf0, buf1, acc,
             gsem0, gsem1, ssem0, ssem1):
    cid = lax.axis_index("c")
    sid = lax.axis_index("s")
    wid = sid * NUM_CORES + cid

    z = jnp.zeros((16,), jnp.float32)

    def _zero_body(i, _):
        for j in range(D_ // 16):
            buf0[i, pl.ds(j * 16, 16)] = z
        return 0

    lax.fori_loop(0, CHUNK, _zero_body, 0)
    row_base = sid * ROWS_PER_TILE
    for k in range(ROWS_PER_TILE // CHUNK):
        pltpu.sync_copy(buf0, acc.at[pl.ds(row_base + k * CHUNK, CHUNK)])
    _zrem = ROWS_PER_TILE - (ROWS_PER_TILE // CHUNK) * CHUNK
    pltpu.sync_copy(
        buf0.at[pl.ds(0, _zrem)],
        acc.at[pl.ds(row_base + ROWS_PER_TILE - _zrem, _zrem)],
    )

    @pl.when(sid == NUM_SUBCORES - 1)
    def _zero_tail():
        pltpu.sync_copy(buf0.at[pl.ds(0, TAIL_ROWS)], acc.at[pl.ds(TAIL_BASE, TAIL_ROWS)])

    plsc.subcore_barrier()

    edge_base = wid * EDGES_PER_TILE
    pltpu.sync_copy(row_hbm.at[pl.ds(edge_base, EDGES_PER_TILE)], rowbig)
    pltpu.sync_copy(col_hbm.at[pl.ds(edge_base, EDGES_PER_TILE)], colbig)

    def _stage_idx(ci, colq, rowq):
        for j in range(CHUNK // 16):
            colq[pl.ds(j * 16, 16)] = colbig[pl.ds(ci * CHUNK + j * 16, 16)]
            rowq[pl.ds(j * 16, 16)] = rowbig[pl.ds(ci * CHUNK + j * 16, 16)]

    def _wait_gather(colq, bufq, gsemq):
        pltpu.make_async_copy(x_hbm.at[colq], bufq, gsemq).wait()

    def _wait_scatter(bufq, rowq, ssemq):
        pltpu.make_async_copy(bufq, acc.at[rowq], ssemq).wait()

    _stage_idx(0, colv0, rowv0)
    pltpu.async_copy(x_hbm.at[colv0], buf0, gsem0)
    _stage_idx(1, colv1, rowv1)
    pltpu.async_copy(x_hbm.at[colv1], buf1, gsem1)

    def _chunk_pair(h, _):
        ci = h * 2
        _wait_gather(colv0, buf0, gsem0)
        pltpu.async_copy(buf0, acc.at[rowv0], ssem0, add=True)
        _wait_gather(colv1, buf1, gsem1)
        pltpu.async_copy(buf1, acc.at[rowv1], ssem1, add=True)
        _wait_scatter(buf0, rowv0, ssem0)
        _stage_idx(ci + 2, colv0, rowv0)
        pltpu.async_copy(x_hbm.at[colv0], buf0, gsem0)
        _wait_scatter(buf1, rowv1, ssem1)
        _stage_idx(ci + 3, colv1, rowv1)
        pltpu.async_copy(x_hbm.at[colv1], buf1, gsem1)
        return 0

    lax.fori_loop(0, (CHUNKS_PER_TILE - 3) // 2, _chunk_pair, 0)
    _wait_gather(colv0, buf0, gsem0)
    pltpu.async_copy(buf0, acc.at[rowv0], ssem0, add=True)
    _wait_gather(colv1, buf1, gsem1)
    pltpu.async_copy(buf1, acc.at[rowv1], ssem1, add=True)
    _wait_scatter(buf0, rowv0, ssem0)
    _stage_idx(CHUNKS_PER_TILE - 1, colv0, rowv0)
    pltpu.async_copy(x_hbm.at[colv0], buf0, gsem0)
    _wait_gather(colv0, buf0, gsem0)
    pltpu.async_copy(buf0, acc.at[rowv0], ssem0, add=True)
    _wait_scatter(buf0, rowv0, ssem0)
    _wait_scatter(buf1, rowv1, ssem1)
    plsc.subcore_barrier()

    for k in range(ROWS_PER_TILE // ZCHUNK):
        pltpu.sync_copy(
            acc.at[pl.ds(row_base + k * ZCHUNK, ZCHUNK)],
            out_hbm.at[cid, pl.ds(row_base + k * ZCHUNK, ZCHUNK)],
        )

    @pl.when(sid == NUM_SUBCORES - 1)
    def _write_tail():
        pltpu.sync_copy(
            acc.at[pl.ds(TAIL_BASE, TAIL_ROWS)],
            out_hbm.at[cid, pl.ds(TAIL_BASE, TAIL_ROWS)],
        )


@jax.jit
def _sc_scatter(x, row, col):
    mesh = plsc.VectorSubcoreMesh(core_axis_name="c", subcore_axis_name="s")
    return pl.kernel(
        _sc_body,
        out_type=jax.ShapeDtypeStruct((NUM_CORES, N_NODES_, D_), jnp.float32),
        mesh=mesh,
        scratch_types=[
            pltpu.VMEM((EDGES_PER_TILE,), jnp.int32),
            pltpu.VMEM((EDGES_PER_TILE,), jnp.int32),
            pltpu.VMEM((CHUNK,), jnp.int32),
            pltpu.VMEM((CHUNK,), jnp.int32),
            pltpu.VMEM((CHUNK,), jnp.int32),
            pltpu.VMEM((CHUNK,), jnp.int32),
            pltpu.VMEM((CHUNK, D_), jnp.float32),
            pltpu.VMEM((CHUNK, D_), jnp.float32),
            pltpu.VMEM_SHARED((N_NODES_, D_), jnp.float32),
            pltpu.SemaphoreType.DMA,
            pltpu.SemaphoreType.DMA,
            pltpu.SemaphoreType.DMA,
            pltpu.SemaphoreType.DMA,
        ],
    )(x, row, col)


ROW_BLK = 1000


def _tc_body(p_ref, x_ref, w_ref, b_ref, o_ref):
    s = p_ref[0] + p_ref[1] + x_ref[...]
    o_ref[...] = jnp.dot(s, w_ref[...], preferred_element_type=jnp.float32) + b_ref[...]


@jax.jit
def _tc_finish(partial, x, W, b2):
    grid = N_NODES_ // ROW_BLK
    return pl.pallas_call(
        _tc_body,
        out_shape=jax.ShapeDtypeStruct((N_NODES_, D_), jnp.float32),
        grid=(grid,),
        in_specs=[
            pl.BlockSpec((NUM_CORES, ROW_BLK, D_), lambda i: (0, i, 0)),
            pl.BlockSpec((ROW_BLK, D_), lambda i: (i, 0)),
            pl.BlockSpec((D_, D_), lambda i: (0, 0)),
            pl.BlockSpec((1, D_), lambda i: (0, 0)),
        ],
        out_specs=pl.BlockSpec((ROW_BLK, D_), lambda i: (i, 0)),
    )(partial, x, W, b2)


def kernel(x, edge_index, W, b):
    ei = edge_index.astype(jnp.int32)
    row = ei[0]
    col = ei[1]
    partial = _sc_scatter(x, row, col)
    return _tc_finish(partial, x, W, b.reshape(1, D_))


# speedup vs baseline: 1.2270x; 1.2270x over previous
import functools

import jax
import jax.numpy as jnp
from jax import lax
from jax.experimental import pallas as pl
from jax.experimental.pallas import tpu as pltpu
from jax.experimental.pallas import tpu_sc as plsc

N_NODES_ = 10000
N_EDGES_ = 320000
D_ = 128

NUM_CORES = 2
NUM_SUBCORES = 16
NUM_TILES = NUM_CORES * NUM_SUBCORES
EDGES_PER_TILE = N_EDGES_ // NUM_TILES
CHUNK = 80
CHUNKS_PER_TILE = EDGES_PER_TILE // CHUNK
ROWS_PER_TILE = 624
ZCHUNK = 208
TAIL_BASE = ROWS_PER_TILE * NUM_SUBCORES
TAIL_ROWS = N_NODES_ - TAIL_BASE


def _sc_body(x_hbm, row_hbm, col_hbm, out_hbm, colbig, rowbig,
             colv0, colv1, rowv0, rowv1, buf0, buf1, acc, gsem0, gsem1):
    cid = lax.axis_index("c")
    sid = lax.axis_index("s")
    wid = sid * NUM_CORES + cid

    z = jnp.zeros((16,), jnp.float32)

    def _zero_body(i, _):
        for j in range(D_ // 16):
            buf0[i, pl.ds(j * 16, 16)] = z
        return 0

    lax.fori_loop(0, CHUNK, _zero_body, 0)
    row_base = sid * ROWS_PER_TILE
    for k in range(ROWS_PER_TILE // CHUNK):
        pltpu.sync_copy(buf0, acc.at[pl.ds(row_base + k * CHUNK, CHUNK)])
    _zrem = ROWS_PER_TILE - (ROWS_PER_TILE // CHUNK) * CHUNK
    pltpu.sync_copy(
        buf0.at[pl.ds(0, _zrem)],
        acc.at[pl.ds(row_base + ROWS_PER_TILE - _zrem, _zrem)],
    )

    @pl.when(sid == NUM_SUBCORES - 1)
    def _zero_tail():
        pltpu.sync_copy(buf0.at[pl.ds(0, TAIL_ROWS)], acc.at[pl.ds(TAIL_BASE, TAIL_ROWS)])

    plsc.subcore_barrier()

    edge_base = wid * EDGES_PER_TILE
    pltpu.sync_copy(row_hbm.at[pl.ds(edge_base, EDGES_PER_TILE)], rowbig)
    pltpu.sync_copy(col_hbm.at[pl.ds(edge_base, EDGES_PER_TILE)], colbig)

    def _stage_idx(ci, colq, rowq):
        for j in range(CHUNK // 16):
            colq[pl.ds(j * 16, 16)] = colbig[pl.ds(ci * CHUNK + j * 16, 16)]
            rowq[pl.ds(j * 16, 16)] = rowbig[pl.ds(ci * CHUNK + j * 16, 16)]

    def _wait_gather(colq, bufq, gsemq):
        pltpu.make_async_copy(x_hbm.at[colq], bufq, gsemq).wait()

    _stage_idx(0, colv0, rowv0)
    pltpu.async_copy(x_hbm.at[colv0], buf0, gsem0)
    _stage_idx(1, colv1, rowv1)
    pltpu.async_copy(x_hbm.at[colv1], buf1, gsem1)

    def _chunk_pair(h, _):
        ci = h * 2
        _wait_gather(colv0, buf0, gsem0)
        pltpu.sync_copy(buf0, acc.at[rowv0], add=True)
        _stage_idx(ci + 2, colv0, rowv0)
        pltpu.async_copy(x_hbm.at[colv0], buf0, gsem0)
        _wait_gather(colv1, buf1, gsem1)
        pltpu.sync_copy(buf1, acc.at[rowv1], add=True)
        _stage_idx(ci + 3, colv1, rowv1)
        pltpu.async_copy(x_hbm.at[colv1], buf1, gsem1)
        return 0

    lax.fori_loop(0, (CHUNKS_PER_TILE - 3) // 2, _chunk_pair, 0)
    _wait_gather(colv0, buf0, gsem0)
    pltpu.sync_copy(buf0, acc.at[rowv0], add=True)
    _stage_idx(CHUNKS_PER_TILE - 1, colv0, rowv0)
    pltpu.async_copy(x_hbm.at[colv0], buf0, gsem0)
    _wait_gather(colv1, buf1, gsem1)
    pltpu.sync_copy(buf1, acc.at[rowv1], add=True)
    _wait_gather(colv0, buf0, gsem0)
    pltpu.sync_copy(buf0, acc.at[rowv0], add=True)
    plsc.subcore_barrier()

    for k in range(ROWS_PER_TILE // ZCHUNK):
        pltpu.sync_copy(
            acc.at[pl.ds(row_base + k * ZCHUNK, ZCHUNK)],
            out_hbm.at[cid, pl.ds(row_base + k * ZCHUNK, ZCHUNK)],
        )

    @pl.when(sid == NUM_SUBCORES - 1)
    def _write_tail():
        pltpu.sync_copy(
            acc.at[pl.ds(TAIL_BASE, TAIL_ROWS)],
            out_hbm.at[cid, pl.ds(TAIL_BASE, TAIL_ROWS)],
        )


@jax.jit
def _sc_scatter(x, row, col):
    mesh = plsc.VectorSubcoreMesh(core_axis_name="c", subcore_axis_name="s")
    return pl.kernel(
        _sc_body,
        out_type=jax.ShapeDtypeStruct((NUM_CORES, N_NODES_, D_), jnp.float32),
        mesh=mesh,
        scratch_types=[
            pltpu.VMEM((EDGES_PER_TILE,), jnp.int32),
            pltpu.VMEM((EDGES_PER_TILE,), jnp.int32),
            pltpu.VMEM((CHUNK,), jnp.int32),
            pltpu.VMEM((CHUNK,), jnp.int32),
            pltpu.VMEM((CHUNK,), jnp.int32),
            pltpu.VMEM((CHUNK,), jnp.int32),
            pltpu.VMEM((CHUNK, D_), jnp.float32),
            pltpu.VMEM((CHUNK, D_), jnp.float32),
            pltpu.VMEM_SHARED((N_NODES_, D_), jnp.float32),
            pltpu.SemaphoreType.DMA,
            pltpu.SemaphoreType.DMA,
        ],
    )(x, row, col)


ROW_BLK = 1000


def _tc_body(p_ref, x_ref, w_ref, b_ref, o_ref):
    s = p_ref[0] + p_ref[1] + x_ref[...]
    o_ref[...] = jnp.dot(s, w_ref[...], preferred_element_type=jnp.float32) + b_ref[...]


@jax.jit
def _tc_finish(partial, x, W, b2):
    grid = N_NODES_ // ROW_BLK
    return pl.pallas_call(
        _tc_body,
        out_shape=jax.ShapeDtypeStruct((N_NODES_, D_), jnp.float32),
        grid=(grid,),
        in_specs=[
            pl.BlockSpec((NUM_CORES, ROW_BLK, D_), lambda i: (0, i, 0)),
            pl.BlockSpec((ROW_BLK, D_), lambda i: (i, 0)),
            pl.BlockSpec((D_, D_), lambda i: (0, 0)),
            pl.BlockSpec((1, D_), lambda i: (0, 0)),
        ],
        out_specs=pl.BlockSpec((ROW_BLK, D_), lambda i: (i, 0)),
    )(partial, x, W, b2)


def kernel(x, edge_index, W, b):
    ei = edge_index.astype(jnp.int32)
    row = ei[0]
    col = ei[1]
    partial = _sc_scatter(x, row, col)
    return _tc_finish(partial, x, W, b.reshape(1, D_))


# speedup vs baseline: 1.3333x; 1.0866x over previous
import functools

import jax
import jax.numpy as jnp
from jax import lax
from jax.experimental import pallas as pl
from jax.experimental.pallas import tpu as pltpu
from jax.experimental.pallas import tpu_sc as plsc

N_NODES_ = 10000
N_EDGES_ = 320000
D_ = 128

NUM_CORES = 2
NUM_SUBCORES = 16
NUM_TILES = NUM_CORES * NUM_SUBCORES
EDGES_PER_TILE = N_EDGES_ // NUM_TILES
CHUNK = 128
FULL_CHUNKS = EDGES_PER_TILE // CHUNK
TAIL_E = EDGES_PER_TILE - FULL_CHUNKS * CHUNK
ROWS_PER_TILE = 624
ZCHUNK = 208
TAIL_BASE = ROWS_PER_TILE * NUM_SUBCORES
TAIL_ROWS = N_NODES_ - TAIL_BASE


def _sc_body(x_hbm, packed_hbm, out_hbm, packedbig,
             colv0, colv1, rowv0, rowv1, tailcol, tailrow,
             buf0, buf1, tailbuf, acc, gsem0, gsem1, tsem):
    cid = lax.axis_index("c")
    sid = lax.axis_index("s")
    wid = sid * NUM_CORES + cid

    z = jnp.zeros((16,), jnp.float32)

    def _zero_body(i, _):
        for j in range(D_ // 16):
            buf0[i, pl.ds(j * 16, 16)] = z
        return 0

    lax.fori_loop(0, CHUNK, _zero_body, 0)
    row_base = sid * ROWS_PER_TILE
    for k in range(ROWS_PER_TILE // CHUNK):
        pltpu.sync_copy(buf0, acc.at[pl.ds(row_base + k * CHUNK, CHUNK)])
    _zrem = ROWS_PER_TILE - (ROWS_PER_TILE // CHUNK) * CHUNK
    pltpu.sync_copy(
        buf0.at[pl.ds(0, _zrem)],
        acc.at[pl.ds(row_base + ROWS_PER_TILE - _zrem, _zrem)],
    )

    @pl.when(sid == NUM_SUBCORES - 1)
    def _zero_tail():
        pltpu.sync_copy(buf0.at[pl.ds(0, TAIL_ROWS)], acc.at[pl.ds(TAIL_BASE, TAIL_ROWS)])

    plsc.subcore_barrier()

    edge_base = wid * EDGES_PER_TILE
    pltpu.sync_copy(packed_hbm.at[pl.ds(edge_base, EDGES_PER_TILE)], packedbig)

    mask16 = jnp.full((16,), 0xFFFF, jnp.int32)

    def _stage_idx(base_e, n, colq, rowq):
        for j in range(n // 16):
            p = packedbig[pl.ds(base_e + j * 16, 16)]
            colq[pl.ds(j * 16, 16)] = lax.bitwise_and(p, mask16)
            rowq[pl.ds(j * 16, 16)] = lax.shift_right_logical(p, 16)

    def _wait_gather(colq, bufq, gsemq):
        pltpu.make_async_copy(x_hbm.at[colq], bufq, gsemq).wait()

    _stage_idx(FULL_CHUNKS * CHUNK, TAIL_E, tailcol, tailrow)
    pltpu.async_copy(x_hbm.at[tailcol], tailbuf, tsem)
    _stage_idx(0, CHUNK, colv0, rowv0)
    pltpu.async_copy(x_hbm.at[colv0], buf0, gsem0)
    _stage_idx(CHUNK, CHUNK, colv1, rowv1)
    pltpu.async_copy(x_hbm.at[colv1], buf1, gsem1)

    def _chunk_pair(h, _):
        ci = h * 2
        _wait_gather(colv0, buf0, gsem0)
        pltpu.sync_copy(buf0, acc.at[rowv0], add=True)
        _stage_idx((ci + 2) * CHUNK, CHUNK, colv0, rowv0)
        pltpu.async_copy(x_hbm.at[colv0], buf0, gsem0)
        _wait_gather(colv1, buf1, gsem1)
        pltpu.sync_copy(buf1, acc.at[rowv1], add=True)
        _stage_idx((ci + 3) * CHUNK, CHUNK, colv1, rowv1)
        pltpu.async_copy(x_hbm.at[colv1], buf1, gsem1)
        return 0

    lax.fori_loop(0, (FULL_CHUNKS - 2) // 2, _chunk_pair, 0)
    _wait_gather(colv0, buf0, gsem0)
    pltpu.sync_copy(buf0, acc.at[rowv0], add=True)
    _wait_gather(colv1, buf1, gsem1)
    pltpu.sync_copy(buf1, acc.at[rowv1], add=True)
    pltpu.make_async_copy(x_hbm.at[tailcol], tailbuf, tsem).wait()
    pltpu.sync_copy(tailbuf, acc.at[tailrow], add=True)
    plsc.subcore_barrier()

    for k in range(ROWS_PER_TILE // ZCHUNK):
        pltpu.sync_copy(
            acc.at[pl.ds(row_base + k * ZCHUNK, ZCHUNK)],
            out_hbm.at[cid, pl.ds(row_base + k * ZCHUNK, ZCHUNK)],
        )

    @pl.when(sid == NUM_SUBCORES - 1)
    def _write_tail():
        pltpu.sync_copy(
            acc.at[pl.ds(TAIL_BASE, TAIL_ROWS)],
            out_hbm.at[cid, pl.ds(TAIL_BASE, TAIL_ROWS)],
        )


@jax.jit
def _sc_scatter(x, packed):
    mesh = plsc.VectorSubcoreMesh(core_axis_name="c", subcore_axis_name="s")
    return pl.kernel(
        _sc_body,
        out_type=jax.ShapeDtypeStruct((NUM_CORES, N_NODES_, D_), jnp.float32),
        mesh=mesh,
        scratch_types=[
            pltpu.VMEM((EDGES_PER_TILE,), jnp.int32),
            pltpu.VMEM((CHUNK,), jnp.int32),
            pltpu.VMEM((CHUNK,), jnp.int32),
            pltpu.VMEM((CHUNK,), jnp.int32),
            pltpu.VMEM((CHUNK,), jnp.int32),
            pltpu.VMEM((TAIL_E,), jnp.int32),
            pltpu.VMEM((TAIL_E,), jnp.int32),
            pltpu.VMEM((CHUNK, D_), jnp.float32),
            pltpu.VMEM((CHUNK, D_), jnp.float32),
            pltpu.VMEM((TAIL_E, D_), jnp.float32),
            pltpu.VMEM_SHARED((N_NODES_, D_), jnp.float32),
            pltpu.SemaphoreType.DMA,
            pltpu.SemaphoreType.DMA,
            pltpu.SemaphoreType.DMA,
        ],
    )(x, packed)


ROW_BLK = 1000


def _tc_body(p_ref, x_ref, w_ref, b_ref, o_ref):
    s = p_ref[0] + p_ref[1] + x_ref[...]
    o_ref[...] = jnp.dot(s, w_ref[...], preferred_element_type=jnp.float32) + b_ref[...]


@jax.jit
def _tc_finish(partial, x, W, b2):
    grid = N_NODES_ // ROW_BLK
    return pl.pallas_call(
        _tc_body,
        out_shape=jax.ShapeDtypeStruct((N_NODES_, D_), jnp.float32),
        grid=(grid,),
        in_specs=[
            pl.BlockSpec((NUM_CORES, ROW_BLK, D_), lambda i: (0, i, 0)),
            pl.BlockSpec((ROW_BLK, D_), lambda i: (i, 0)),
            pl.BlockSpec((D_, D_), lambda i: (0, 0)),
            pl.BlockSpec((1, D_), lambda i: (0, 0)),
        ],
        out_specs=pl.BlockSpec((ROW_BLK, D_), lambda i: (i, 0)),
    )(partial, x, W, b2)


def kernel(x, edge_index, W, b):
    ei = edge_index.astype(jnp.int32)
    packed = jnp.bitwise_or(jnp.left_shift(ei[0], 16), ei[1])
    partial = _sc_scatter(x, packed)
    return _tc_finish(partial, x, W, b.reshape(1, D_))


# speedup vs baseline: 1.3893x; 1.0420x over previous
import functools

import jax
import jax.numpy as jnp
from jax import lax
from jax.experimental import pallas as pl
from jax.experimental.pallas import tpu as pltpu
from jax.experimental.pallas import tpu_sc as plsc

N_NODES_ = 10000
N_EDGES_ = 320000
D_ = 128

NUM_CORES = 2
NUM_SUBCORES = 16
NUM_TILES = NUM_CORES * NUM_SUBCORES
EDGES_PER_TILE = N_EDGES_ // NUM_TILES
CHUNK = 96
FULL_CHUNKS = EDGES_PER_TILE // CHUNK
TAIL_E = EDGES_PER_TILE - FULL_CHUNKS * CHUNK
ROWS_PER_TILE = 624
ZCHUNK = 208
TAIL_BASE = ROWS_PER_TILE * NUM_SUBCORES
TAIL_ROWS = N_NODES_ - TAIL_BASE


def _sc_body(x_hbm, flat_hbm, out_hbm, colbig, rowbig,
             colv0, colv1, rowv0, rowv1, tailcol, tailrow,
             buf0, buf1, tailbuf, acc, gsem0, gsem1, tsem):
    cid = lax.axis_index("c")
    sid = lax.axis_index("s")
    wid = sid * NUM_CORES + cid

    z = jnp.zeros((16,), jnp.float32)

    def _zero_body(i, _):
        for j in range(D_ // 16):
            buf0[i, pl.ds(j * 16, 16)] = z
        return 0

    lax.fori_loop(0, CHUNK, _zero_body, 0)
    row_base = sid * ROWS_PER_TILE
    for k in range(ROWS_PER_TILE // CHUNK):
        pltpu.sync_copy(buf0, acc.at[pl.ds(row_base + k * CHUNK, CHUNK)])
    _zrem = ROWS_PER_TILE - (ROWS_PER_TILE // CHUNK) * CHUNK
    pltpu.sync_copy(
        buf0.at[pl.ds(0, _zrem)],
        acc.at[pl.ds(row_base + ROWS_PER_TILE - _zrem, _zrem)],
    )

    @pl.when(sid == NUM_SUBCORES - 1)
    def _zero_tail():
        pltpu.sync_copy(buf0.at[pl.ds(0, TAIL_ROWS)], acc.at[pl.ds(TAIL_BASE, TAIL_ROWS)])

    plsc.subcore_barrier()

    edge_base = wid * EDGES_PER_TILE
    pltpu.sync_copy(flat_hbm.at[pl.ds(edge_base, EDGES_PER_TILE)], rowbig)
    pltpu.sync_copy(flat_hbm.at[pl.ds(N_EDGES_ + edge_base, EDGES_PER_TILE)], colbig)

    def _stage_idx(base_e, n, colq, rowq):
        for j in range(n // 16):
            colq[pl.ds(j * 16, 16)] = colbig[pl.ds(base_e + j * 16, 16)]
            rowq[pl.ds(j * 16, 16)] = rowbig[pl.ds(base_e + j * 16, 16)]

    def _wait_gather(colq, bufq, gsemq):
        pltpu.make_async_copy(x_hbm.at[colq], bufq, gsemq).wait()

    _stage_idx(FULL_CHUNKS * CHUNK, TAIL_E, tailcol, tailrow)
    pltpu.async_copy(x_hbm.at[tailcol], tailbuf, tsem)
    _stage_idx(0, CHUNK, colv0, rowv0)
    pltpu.async_copy(x_hbm.at[colv0], buf0, gsem0)
    _stage_idx(CHUNK, CHUNK, colv1, rowv1)
    pltpu.async_copy(x_hbm.at[colv1], buf1, gsem1)

    def _chunk_pair(h, _):
        ci = h * 2
        _wait_gather(colv0, buf0, gsem0)
        pltpu.sync_copy(buf0, acc.at[rowv0], add=True)
        _stage_idx((ci + 2) * CHUNK, CHUNK, colv0, rowv0)
        pltpu.async_copy(x_hbm.at[colv0], buf0, gsem0)
        _wait_gather(colv1, buf1, gsem1)
        pltpu.sync_copy(buf1, acc.at[rowv1], add=True)
        _stage_idx((ci + 3) * CHUNK, CHUNK, colv1, rowv1)
        pltpu.async_copy(x_hbm.at[colv1], buf1, gsem1)
        return 0

    lax.fori_loop(0, (FULL_CHUNKS - 2) // 2, _chunk_pair, 0)
    _wait_gather(colv0, buf0, gsem0)
    pltpu.sync_copy(buf0, acc.at[rowv0], add=True)
    _wait_gather(colv1, buf1, gsem1)
    pltpu.sync_copy(buf1, acc.at[rowv1], add=True)
    pltpu.make_async_copy(x_hbm.at[tailcol], tailbuf, tsem).wait()
    pltpu.sync_copy(tailbuf, acc.at[tailrow], add=True)
    plsc.subcore_barrier()

    for k in range(ROWS_PER_TILE // ZCHUNK):
        pltpu.sync_copy(
            acc.at[pl.ds(row_base + k * ZCHUNK, ZCHUNK)],
            out_hbm.at[cid, pl.ds(row_base + k * ZCHUNK, ZCHUNK)],
        )

    @pl.when(sid == NUM_SUBCORES - 1)
    def _write_tail():
        pltpu.sync_copy(
            acc.at[pl.ds(TAIL_BASE, TAIL_ROWS)],
            out_hbm.at[cid, pl.ds(TAIL_BASE, TAIL_ROWS)],
        )


@jax.jit
def _sc_scatter(x, flat):
    mesh = plsc.VectorSubcoreMesh(core_axis_name="c", subcore_axis_name="s")
    return pl.kernel(
        _sc_body,
        out_type=jax.ShapeDtypeStruct((NUM_CORES, N_NODES_, D_), jnp.float32),
        mesh=mesh,
        scratch_types=[
            pltpu.VMEM((EDGES_PER_TILE,), jnp.int32),
            pltpu.VMEM((EDGES_PER_TILE,), jnp.int32),
            pltpu.VMEM((CHUNK,), jnp.int32),
            pltpu.VMEM((CHUNK,), jnp.int32),
            pltpu.VMEM((CHUNK,), jnp.int32),
            pltpu.VMEM((CHUNK,), jnp.int32),
            pltpu.VMEM((TAIL_E,), jnp.int32),
            pltpu.VMEM((TAIL_E,), jnp.int32),
            pltpu.VMEM((CHUNK, D_), jnp.float32),
            pltpu.VMEM((CHUNK, D_), jnp.float32),
            pltpu.VMEM((TAIL_E, D_), jnp.float32),
            pltpu.VMEM_SHARED((N_NODES_, D_), jnp.float32),
            pltpu.SemaphoreType.DMA,
            pltpu.SemaphoreType.DMA,
            pltpu.SemaphoreType.DMA,
        ],
    )(x, flat)


ROW_BLK = 2000


def _tc_body(p_ref, x_ref, w_ref, b_ref, o_ref):
    s = p_ref[0] + p_ref[1] + x_ref[...]
    o_ref[...] = jnp.dot(s, w_ref[...], preferred_element_type=jnp.float32) + b_ref[...]


@jax.jit
def _tc_finish(partial, x, W, b2):
    grid = N_NODES_ // ROW_BLK
    return pl.pallas_call(
        _tc_body,
        out_shape=jax.ShapeDtypeStruct((N_NODES_, D_), jnp.float32),
        grid=(grid,),
        in_specs=[
            pl.BlockSpec((NUM_CORES, ROW_BLK, D_), lambda i: (0, i, 0)),
            pl.BlockSpec((ROW_BLK, D_), lambda i: (i, 0)),
            pl.BlockSpec((D_, D_), lambda i: (0, 0)),
            pl.BlockSpec((1, D_), lambda i: (0, 0)),
        ],
        out_specs=pl.BlockSpec((ROW_BLK, D_), lambda i: (i, 0)),
    )(partial, x, W, b2)


def kernel(x, edge_index, W, b):
    flat = edge_index.astype(jnp.int32).reshape(2 * N_EDGES_)
    partial = _sc_scatter(x, flat)
    return _tc_finish(partial, x, W, b.reshape(1, D_))


# speedup vs baseline: 1.4641x; 1.0538x over previous
import jax
import jax.numpy as jnp
from jax import lax
from jax.experimental import pallas as pl
from jax.experimental.pallas import tpu as pltpu
from jax.experimental.pallas import tpu_sc as plsc

N_NODES_ = 10000
N_EDGES_ = 320000
D_ = 128

NUM_CORES = 2
NUM_SUBCORES = 16
NUM_TILES = NUM_CORES * NUM_SUBCORES
CHUNK = 128
NCHUNKS = N_EDGES_ // CHUNK
STEPS = NCHUNKS // NUM_TILES
EXTRA_CHUNKS = NCHUNKS - STEPS * NUM_TILES
ROWS_PER_TILE = 624
ZCHUNK = 208
TAIL_BASE = ROWS_PER_TILE * NUM_SUBCORES
TAIL_ROWS = N_NODES_ - TAIL_BASE


def _sc_body(x_hbm, flat_hbm, out_hbm,
             colv0, colv1, colv2, colv3, rowv0, rowv1, rowv2, rowv3,
             buf0, buf1, acc,
             gsem0, gsem1, isem0, isem1, isem2, isem3):
    cid = lax.axis_index("c")
    sid = lax.axis_index("s")
    wid = sid * NUM_CORES + cid

    colv = (colv0, colv1, colv2, colv3)
    rowv = (rowv0, rowv1, rowv2, rowv3)
    isem = (isem0, isem1, isem2, isem3)
    bufs = (buf0, buf1)
    gsem = (gsem0, gsem1)

    z = jnp.zeros((16,), jnp.float32)

    def _zero_body(i, _):
        for j in range(D_ // 16):
            buf0[i, pl.ds(j * 16, 16)] = z
        return 0

    lax.fori_loop(0, CHUNK, _zero_body, 0)
    row_base = sid * ROWS_PER_TILE
    for k in range(ROWS_PER_TILE // CHUNK):
        pltpu.sync_copy(buf0, acc.at[pl.ds(row_base + k * CHUNK, CHUNK)])
    _zrem = ROWS_PER_TILE - (ROWS_PER_TILE // CHUNK) * CHUNK
    pltpu.sync_copy(
        buf0.at[pl.ds(0, _zrem)],
        acc.at[pl.ds(row_base + ROWS_PER_TILE - _zrem, _zrem)],
    )

    @pl.when(sid == NUM_SUBCORES - 1)
    def _zero_tail():
        pltpu.sync_copy(buf0.at[pl.ds(0, TAIL_ROWS)], acc.at[pl.ds(TAIL_BASE, TAIL_ROWS)])

    plsc.subcore_barrier()

    def _issue_idx(c, p):
        base = c * CHUNK
        pltpu.async_copy(flat_hbm.at[pl.ds(base, CHUNK)], rowv[p], isem[p])
        pltpu.async_copy(flat_hbm.at[pl.ds(N_EDGES_ + base, CHUNK)], colv[p], isem[p])

    def _wait_idx(c, p):
        base = c * CHUNK
        pltpu.make_async_copy(flat_hbm.at[pl.ds(base, CHUNK)], rowv[p], isem[p]).wait()
        pltpu.make_async_copy(flat_hbm.at[pl.ds(N_EDGES_ + base, CHUNK)], colv[p], isem[p]).wait()

    def _chunk_of(k):
        return k * NUM_TILES + wid

    def _wait_gather(p, d):
        pltpu.make_async_copy(x_hbm.at[colv[p]], bufs[d], gsem[d]).wait()

    for p in range(4):
        _issue_idx(_chunk_of(p), p)
    _wait_idx(_chunk_of(0), 0)
    pltpu.async_copy(x_hbm.at[colv[0]], bufs[0], gsem[0])
    _wait_idx(_chunk_of(1), 1)
    pltpu.async_copy(x_hbm.at[colv[1]], bufs[1], gsem[1])

    def _quad(q, _):
        for b in range(4):
            k = 4 * q + b
            d = b % 2
            _wait_gather(b, d)
            pltpu.sync_copy(bufs[d], acc.at[rowv[b]], add=True)

            @pl.when(k + 4 <= STEPS - 1)
            def _prefetch_idx():
                _issue_idx(_chunk_of(k + 4), b)

            p2 = (b + 2) % 4
            _wait_idx(_chunk_of(k + 2), p2)
            pltpu.async_copy(x_hbm.at[colv[p2]], bufs[d], gsem[d])
        return 0

    lax.fori_loop(0, (STEPS - 2) // 4, _quad, 0)
    _wait_gather(0, 0)
    pltpu.sync_copy(bufs[0], acc.at[rowv[0]], add=True)
    _wait_gather(1, 1)
    pltpu.sync_copy(bufs[1], acc.at[rowv[1]], add=True)

    @pl.when(wid < EXTRA_CHUNKS)
    def _extra():
        c = STEPS * NUM_TILES + wid
        _issue_idx(c, 2)
        _wait_idx(c, 2)
        pltpu.async_copy(x_hbm.at[colv[2]], bufs[0], gsem[0])
        _wait_gather(2, 0)
        pltpu.sync_copy(bufs[0], acc.at[rowv[2]], add=True)

    plsc.subcore_barrier()

    for k in range(ROWS_PER_TILE // ZCHUNK):
        pltpu.sync_copy(
            acc.at[pl.ds(row_base + k * ZCHUNK, ZCHUNK)],
            out_hbm.at[cid, pl.ds(row_base + k * ZCHUNK, ZCHUNK)],
        )

    @pl.when(sid == NUM_SUBCORES - 1)
    def _write_tail():
        pltpu.sync_copy(
            acc.at[pl.ds(TAIL_BASE, TAIL_ROWS)],
            out_hbm.at[cid, pl.ds(TAIL_BASE, TAIL_ROWS)],
        )


@jax.jit
def _sc_scatter(x, flat):
    mesh = plsc.VectorSubcoreMesh(core_axis_name="c", subcore_axis_name="s")
    return pl.kernel(
        _sc_body,
        out_type=jax.ShapeDtypeStruct((NUM_CORES, N_NODES_, D_), jnp.float32),
        mesh=mesh,
        scratch_types=[
            pltpu.VMEM((CHUNK,), jnp.int32),
            pltpu.VMEM((CHUNK,), jnp.int32),
            pltpu.VMEM((CHUNK,), jnp.int32),
            pltpu.VMEM((CHUNK,), jnp.int32),
            pltpu.VMEM((CHUNK,), jnp.int32),
            pltpu.VMEM((CHUNK,), jnp.int32),
            pltpu.VMEM((CHUNK,), jnp.int32),
            pltpu.VMEM((CHUNK,), jnp.int32),
            pltpu.VMEM((CHUNK, D_), jnp.float32),
            pltpu.VMEM((CHUNK, D_), jnp.float32),
            pltpu.VMEM_SHARED((N_NODES_, D_), jnp.float32),
            pltpu.SemaphoreType.DMA,
            pltpu.SemaphoreType.DMA,
            pltpu.SemaphoreType.DMA,
            pltpu.SemaphoreType.DMA,
            pltpu.SemaphoreType.DMA,
            pltpu.SemaphoreType.DMA,
        ],
    )(x, flat)


ROW_BLK = 2000


def _tc_body(p_ref, x_ref, w_ref, b_ref, o_ref):
    s = p_ref[0] + p_ref[1] + x_ref[...]
    o_ref[...] = jnp.dot(s, w_ref[...], preferred_element_type=jnp.float32) + b_ref[...]


@jax.jit
def _tc_finish(partial, x, W, b2):
    grid = N_NODES_ // ROW_BLK
    return pl.pallas_call(
        _tc_body,
        out_shape=jax.ShapeDtypeStruct((N_NODES_, D_), jnp.float32),
        grid=(grid,),
        in_specs=[
            pl.BlockSpec((NUM_CORES, ROW_BLK, D_), lambda i: (0, i, 0)),
            pl.BlockSpec((ROW_BLK, D_), lambda i: (i, 0)),
            pl.BlockSpec((D_, D_), lambda i: (0, 0)),
            pl.BlockSpec((1, D_), lambda i: (0, 0)),
        ],
        out_specs=pl.BlockSpec((ROW_BLK, D_), lambda i: (i, 0)),
    )(partial, x, W, b2)


def kernel(x, edge_index, W, b):
    flat = edge_index.astype(jnp.int32).reshape(2 * N_EDGES_)
    partial = _sc_scatter(x, flat)
    return _tc_finish(partial, x, W, b.reshape(1, D_))


# speedup vs baseline: 1.6119x; 1.1010x over previous
import jax
import jax.numpy as jnp
from jax import lax
from jax.experimental import pallas as pl
from jax.experimental.pallas import tpu as pltpu
from jax.experimental.pallas import tpu_sc as plsc

N_NODES_ = 10000
N_EDGES_ = 320000
D_ = 128

NUM_CORES = 2
NUM_SUBCORES = 16
NUM_TILES = NUM_CORES * NUM_SUBCORES
CHUNK = 128
NCHUNKS = N_EDGES_ // CHUNK
STEPS = NCHUNKS // NUM_TILES
EXTRA_CHUNKS = NCHUNKS - STEPS * NUM_TILES
ROWS_PER_TILE = 624
ZCHUNK = 208
TAIL_BASE = ROWS_PER_TILE * NUM_SUBCORES
TAIL_ROWS = N_NODES_ - TAIL_BASE


NBUF = 3
NIDX = 6


def _sc_body(x_hbm, flat_hbm, out_hbm,
             colv0, colv1, colv2, colv3, colv4, colv5,
             rowv0, rowv1, rowv2, rowv3, rowv4, rowv5,
             buf0, buf1, buf2, acc,
             gsem0, gsem1, gsem2,
             isem0, isem1, isem2, isem3, isem4, isem5):
    cid = lax.axis_index("c")
    sid = lax.axis_index("s")
    wid = sid * NUM_CORES + cid

    colv = (colv0, colv1, colv2, colv3, colv4, colv5)
    rowv = (rowv0, rowv1, rowv2, rowv3, rowv4, rowv5)
    isem = (isem0, isem1, isem2, isem3, isem4, isem5)
    bufs = (buf0, buf1, buf2)
    gsem = (gsem0, gsem1, gsem2)

    z = jnp.zeros((16,), jnp.float32)

    def _zero_body(i, _):
        for j in range(D_ // 16):
            buf0[i, pl.ds(j * 16, 16)] = z
        return 0

    lax.fori_loop(0, CHUNK, _zero_body, 0)
    row_base = sid * ROWS_PER_TILE
    for k in range(ROWS_PER_TILE // CHUNK):
        pltpu.sync_copy(buf0, acc.at[pl.ds(row_base + k * CHUNK, CHUNK)])
    _zrem = ROWS_PER_TILE - (ROWS_PER_TILE // CHUNK) * CHUNK
    pltpu.sync_copy(
        buf0.at[pl.ds(0, _zrem)],
        acc.at[pl.ds(row_base + ROWS_PER_TILE - _zrem, _zrem)],
    )

    @pl.when(sid == NUM_SUBCORES - 1)
    def _zero_tail():
        pltpu.sync_copy(buf0.at[pl.ds(0, TAIL_ROWS)], acc.at[pl.ds(TAIL_BASE, TAIL_ROWS)])

    plsc.subcore_barrier()

    def _issue_idx(c, p):
        base = c * CHUNK
        pltpu.async_copy(flat_hbm.at[pl.ds(base, CHUNK)], rowv[p], isem[p])
        pltpu.async_copy(flat_hbm.at[pl.ds(N_EDGES_ + base, CHUNK)], colv[p], isem[p])

    def _wait_idx(c, p):
        base = c * CHUNK
        pltpu.make_async_copy(flat_hbm.at[pl.ds(base, CHUNK)], rowv[p], isem[p]).wait()
        pltpu.make_async_copy(flat_hbm.at[pl.ds(N_EDGES_ + base, CHUNK)], colv[p], isem[p]).wait()

    def _chunk_of(k):
        return k * NUM_TILES + wid

    def _wait_gather(p, d):
        pltpu.make_async_copy(x_hbm.at[colv[p]], bufs[d], gsem[d]).wait()

    for p in range(NIDX):
        _issue_idx(_chunk_of(p), p)
    for d in range(NBUF):
        _wait_idx(_chunk_of(d), d)
        pltpu.async_copy(x_hbm.at[colv[d]], bufs[d], gsem[d])

    def _sextet(t, _):
        for b in range(NIDX):
            k = NIDX * t + b
            d = b % NBUF
            _wait_gather(b, d)
            pltpu.sync_copy(bufs[d], acc.at[rowv[b]], add=True)

            @pl.when(k + NIDX <= STEPS - 1)
            def _prefetch_idx():
                _issue_idx(_chunk_of(k + NIDX), b)

            @pl.when(k + NBUF <= STEPS - 1)
            def _next_gather():
                p2 = (b + NBUF) % NIDX
                _wait_idx(_chunk_of(k + NBUF), p2)
                pltpu.async_copy(x_hbm.at[colv[p2]], bufs[d], gsem[d])
        return 0

    lax.fori_loop(0, STEPS // NIDX, _sextet, 0)

    @pl.when(wid < EXTRA_CHUNKS)
    def _extra():
        c = STEPS * NUM_TILES + wid
        _issue_idx(c, 0)
        _wait_idx(c, 0)
        pltpu.async_copy(x_hbm.at[colv[0]], bufs[0], gsem[0])
        _wait_gather(0, 0)
        pltpu.sync_copy(bufs[0], acc.at[rowv[0]], add=True)

    plsc.subcore_barrier()

    for k in range(ROWS_PER_TILE // ZCHUNK):
        pltpu.sync_copy(
            acc.at[pl.ds(row_base + k * ZCHUNK, ZCHUNK)],
            out_hbm.at[cid, pl.ds(row_base + k * ZCHUNK, ZCHUNK)],
        )

    @pl.when(sid == NUM_SUBCORES - 1)
    def _write_tail():
        pltpu.sync_copy(
            acc.at[pl.ds(TAIL_BASE, TAIL_ROWS)],
            out_hbm.at[cid, pl.ds(TAIL_BASE, TAIL_ROWS)],
        )


@jax.jit
def _sc_scatter(x, flat):
    mesh = plsc.VectorSubcoreMesh(core_axis_name="c", subcore_axis_name="s")
    return pl.kernel(
        _sc_body,
        out_type=jax.ShapeDtypeStruct((NUM_CORES, N_NODES_, D_), jnp.float32),
        mesh=mesh,
        scratch_types=(
            [pltpu.VMEM((CHUNK,), jnp.int32)] * (2 * NIDX)
            + [pltpu.VMEM((CHUNK, D_), jnp.float32)] * NBUF
            + [pltpu.VMEM_SHARED((N_NODES_, D_), jnp.float32)]
            + [pltpu.SemaphoreType.DMA] * (NBUF + NIDX)
        ),
    )(x, flat)


ROW_BLK = 2000


def _tc_body(p_ref, x_ref, w_ref, b_ref, o_ref):
    s = p_ref[0] + p_ref[1] + x_ref[...]
    o_ref[...] = jnp.dot(s, w_ref[...], preferred_element_type=jnp.float32) + b_ref[...]


@jax.jit
def _tc_finish(partial, x, W, b2):
    grid = N_NODES_ // ROW_BLK
    return pl.pallas_call(
        _tc_body,
        out_shape=jax.ShapeDtypeStruct((N_NODES_, D_), jnp.float32),
        grid=(grid,),
        in_specs=[
            pl.BlockSpec((NUM_CORES, ROW_BLK, D_), lambda i: (0, i, 0)),
            pl.BlockSpec((ROW_BLK, D_), lambda i: (i, 0)),
            pl.BlockSpec((D_, D_), lambda i: (0, 0)),
            pl.BlockSpec((1, D_), lambda i: (0, 0)),
        ],
        out_specs=pl.BlockSpec((ROW_BLK, D_), lambda i: (i, 0)),
    )(partial, x, W, b2)


def kernel(x, edge_index, W, b):
    flat = edge_index.astype(jnp.int32).reshape(2 * N_EDGES_)
    partial = _sc_scatter(x, flat)
    return _tc_finish(partial, x, W, b.reshape(1, D_))
